# baseline (speedup 1.0000x reference)
import functools

import jax
import jax.numpy as jnp
from jax import lax
from jax.experimental import pallas as pl
from jax.experimental.pallas import tpu as pltpu
from jax.experimental.pallas import tpu_sc as plsc

N = 10000
E = 320000
D = 128
DE = 144
R = 16

_INFO = plsc.get_sparse_core_info()
NC = _INFO.num_cores
NS = _INFO.num_subcores
NW = NC * NS
B = 64
NBLK = 160
NGRP = NBLK // 4
EW = NBLK * B
EP = NW * EW
NP = 10240
RPT = NP // NS


def _rel_matmul(x, w):
    nb = 5
    bn = N // nb

    def body(x_ref, w_ref, o_ref):
        o_ref[...] = jnp.dot(
            x_ref[...], w_ref[0], preferred_element_type=jnp.float32)

    return pl.pallas_call(
        body,
        grid=(nb, R),
        in_specs=[
            pl.BlockSpec((bn, D), lambda i, r: (i, 0)),
            pl.BlockSpec((1, D, D), lambda i, r: (r, 0, 0)),
        ],
        out_specs=pl.BlockSpec((bn, D), lambda i, r: (r * nb + i, 0)),
        out_shape=jax.ShapeDtypeStruct((R * N, D), jnp.float32),
    )(x, w)


def _finish1(p, deg, x, w, b):
    nb = 5
    bn = N // nb

    def body(p_ref, deg_ref, x_ref, w_ref, b_ref, o_ref, inv_ref):
        s = p_ref[0] + p_ref[1]
        d = deg_ref[0][:, 0:1] + deg_ref[1][:, 0:1]
        inv = 1.0 / jnp.maximum(d, 1.0)
        lin = jnp.dot(x_ref[...], w_ref[...], preferred_element_type=jnp.float32)
        o_ref[...] = jnp.tanh(s * inv + lin + b_ref[...])
        inv_ref[...] = inv

    return pl.pallas_call(
        body,
        grid=(nb,),
        in_specs=[
            pl.BlockSpec((2, bn, D), lambda i: (0, i, 0)),
            pl.BlockSpec((2, bn, 16), lambda i: (0, i, 0)),
            pl.BlockSpec((bn, D), lambda i: (i, 0)),
            pl.BlockSpec((D, D), lambda i: (0, 0)),
            pl.BlockSpec((1, D), lambda i: (0, 0)),
        ],
        out_specs=[
            pl.BlockSpec((bn, D), lambda i: (i, 0)),
            pl.BlockSpec((bn, 1), lambda i: (i, 0)),
        ],
        out_shape=[
            jax.ShapeDtypeStruct((N, D), jnp.float32),
            jax.ShapeDtypeStruct((N, 1), jnp.float32),
        ],
    )(p, deg, x, w, b.reshape(1, D))


def _finish2(p, inv, x, w, b):
    nb = 5
    bn = N // nb

    def body(p_ref, inv_ref, x_ref, w_ref, b_ref, o_ref):
        s = p_ref[0] + p_ref[1]
        lin = jnp.dot(x_ref[...], w_ref[...], preferred_element_type=jnp.float32)
        o_ref[...] = jnp.tanh(s * inv_ref[...] + lin + b_ref[...])

    return pl.pallas_call(
        body,
        grid=(nb,),
        in_specs=[
            pl.BlockSpec((2, bn, D), lambda i: (0, i, 0)),
            pl.BlockSpec((bn, 1), lambda i: (i, 0)),
            pl.BlockSpec((bn, D), lambda i: (i, 0)),
            pl.BlockSpec((D, D), lambda i: (0, 0)),
            pl.BlockSpec((1, D), lambda i: (0, 0)),
        ],
        out_specs=pl.BlockSpec((bn, D), lambda i: (i, 0)),
        out_shape=jax.ShapeDtypeStruct((N, D), jnp.float32),
    )(p, inv, x, w, b.reshape(1, D))


def _make_sc_agg(tc_tiling, with_deg):
    mesh = plsc.VectorSubcoreMesh(core_axis_name="c", subcore_axis_name="s")
    out_type = [jax.ShapeDtypeStruct((NC, NP, D), jnp.float32)]
    scratch = (
        [pltpu.VMEM((2, B), jnp.int32) for _ in range(4)]
        + [pltpu.VMEM((2, B), jnp.int32) for _ in range(4)]
        + [pltpu.VMEM((B, D), jnp.float32) for _ in range(4)]
        + [pltpu.VMEM_SHARED((NP, D), jnp.float32)]
        + [pltpu.SemaphoreType.DMA for _ in range(12)]
    )
    if with_deg:
        out_type.append(jax.ShapeDtypeStruct((NC, NP, 16), jnp.float32))
        scratch.append(pltpu.VMEM((B, 16), jnp.float32))
        scratch.append(pltpu.VMEM_SHARED((NP, 16), jnp.float32))

    def body(table, pairs, zrow, z16, o16, *rest):
        nout = 2 if with_deg else 1
        out = rest[0]
        degout = rest[1] if with_deg else None
        sc = rest[nout:]
        pba = sc[0:4]
        pbb = sc[4:8]
        rows = sc[8:12]
        acc_sh = sc[12]
        ia = sc[13:17]
        ib = sc[17:21]
        gs = sc[21:25]
        ones_v = sc[25] if with_deg else None
        deg_sh = sc[26] if with_deg else None
        c = lax.axis_index("c")
        s = lax.axis_index("s")
        wid = s * NC + c
        r0 = s * RPT
        nchunk = RPT // B

        pltpu.sync_copy(zrow, rows[0])
        for k in range(nchunk):
            pltpu.sync_copy(rows[0], acc_sh.at[pl.ds(r0 + k * B, B)])
        if with_deg:
            pltpu.sync_copy(z16, ones_v)
            for k in range(nchunk):
                pltpu.sync_copy(ones_v, deg_sh.at[pl.ds(r0 + k * B, B)])
            pltpu.sync_copy(o16, ones_v)
        for j in range(4):
            pltpu.sync_copy(pairs.at[wid, j], pba[j])
            pltpu.async_copy(pairs.at[wid, 4 + j], pbb[j], ib[j])
        plsc.subcore_barrier()
        for j in range(4):
            pltpu.async_copy(table.at[pba[j].at[0]], rows[j], gs[j])

        def scat(rj, pbj):
            pltpu.sync_copy(rj, acc_sh.at[pbj.at[1]], add=True)
            if with_deg:
                pltpu.sync_copy(ones_v, deg_sh.at[pbj.at[1]], add=True)

        def two_groups(i, carry):
            g0 = 2 * i
            for j in range(4):
                pltpu.make_async_copy(
                    table.at[pba[j].at[0]], rows[j], gs[j]).wait()
                scat(rows[j], pba[j])

                @pl.when(g0 + 2 < NGRP)
                def _():
                    pltpu.async_copy(
                        pairs.at[wid, (g0 + 2) * 4 + j], pba[j], ia[j])

                pltpu.make_async_copy(
                    pairs.at[wid, (g0 + 1) * 4 + j], pbb[j], ib[j]).wait()
                pltpu.async_copy(table.at[pbb[j].at[0]], rows[j], gs[j])
            for j in range(4):
                pltpu.make_async_copy(
                    table.at[pbb[j].at[0]], rows[j], gs[j]).wait()
                scat(rows[j], pbb[j])

                @pl.when(g0 + 3 < NGRP)
                def _():
                    pltpu.async_copy(
                        pairs.at[wid, (g0 + 3) * 4 + j], pbb[j], ib[j])

                @pl.when(g0 + 2 < NGRP)
                def _():
                    pltpu.make_async_copy(
                        pairs.at[wid, (g0 + 2) * 4 + j], pba[j], ia[j]).wait()
                    pltpu.async_copy(table.at[pba[j].at[0]], rows[j], gs[j])

            return carry

        lax.fori_loop(0, NGRP // 2, two_groups, 0)
        plsc.subcore_barrier()

        for k in range(nchunk):
            pltpu.sync_copy(acc_sh.at[pl.ds(r0 + k * B, B)], rows[0])
            pltpu.sync_copy(rows[0], out.at[c, pl.ds(r0 + k * B, B)])
        if with_deg:
            for k in range(nchunk):
                pltpu.sync_copy(deg_sh.at[pl.ds(r0 + k * B, B)], ones_v)
                pltpu.sync_copy(ones_v, degout.at[c, pl.ds(r0 + k * B, B)])

    return pl.kernel(
        body, out_type=tuple(out_type), mesh=mesh,
        scratch_types=scratch,
        compiler_params=pltpu.CompilerParams(use_tc_tiling_on_sc=tc_tiling))


_SC_AGG_DEG = _make_sc_agg(tc_tiling=False, with_deg=True)
_SC_AGG = _make_sc_agg(tc_tiling=True, with_deg=False)


def kernel(node_feats, edge_index, edge_types, rel_w1, lin_w1, lin_b1,
           rel_w2, lin_w2, lin_b2):
    src = edge_index[0].astype(jnp.int32)
    dst = edge_index[1].astype(jnp.int32)
    et = edge_types.astype(jnp.int32)

    padw = EW - E // NW
    cidx = (et * N + src).reshape(NW, E // NW)
    cidx = jnp.concatenate(
        [cidx, jnp.zeros((NW, padw), jnp.int32)], axis=1)
    fill = N + (jnp.arange(NW * padw, dtype=jnp.int32).reshape(NW, padw)
                % (NP - N))
    dstp = jnp.concatenate([dst.reshape(NW, E // NW), fill], axis=1)
    pairs3 = jnp.stack(
        [cidx.reshape(NW, NBLK, B), dstp.reshape(NW, NBLK, B)], axis=2)

    zD = jnp.zeros((B, D), jnp.float32)
    z16 = jnp.zeros((B, 16), jnp.float32)
    o16 = jnp.ones((B, 16), jnp.float32)

    hr1 = _rel_matmul(node_feats, rel_w1)
    p1, deg = _SC_AGG_DEG(hr1, pairs3, zD, z16, o16)
    h1, inv = _finish1(p1, deg, node_feats, lin_w1, lin_b1)

    hr2 = _rel_matmul(h1, rel_w2)
    (p2,) = _SC_AGG(hr2, pairs3, zD, z16, o16)
    h2 = _finish2(p2, inv, h1, lin_w2, lin_b2)
    return h2
